# baseline (speedup 1.0000x reference)
import functools

import jax
import jax.numpy as jnp
from jax import lax
from jax.experimental import pallas as pl
from jax.experimental.pallas import tpu as pltpu
from jax.experimental.pallas import tpu_sc as plsc

_D = 64
_CHUNK = 128


@functools.lru_cache(maxsize=None)
def _build(batch: int):
    info = plsc.get_sparse_core_info()
    nw = info.num_cores * info.num_subcores
    bpw = batch // nw
    nch = bpw // _CHUNK
    mesh = plsc.VectorSubcoreMesh(core_axis_name="c", subcore_axis_name="s")

    @functools.partial(
        pl.kernel,
        mesh=mesh,
        out_type=(
            jax.ShapeDtypeStruct((batch, _D), jnp.float32),
            jax.ShapeDtypeStruct((batch, _D), jnp.float32),
        ),
        scratch_types=[
            pltpu.VMEM((bpw,), jnp.int32),
            pltpu.VMEM((bpw,), jnp.int32),
            pltpu.VMEM((bpw, _D), jnp.float32),
            pltpu.VMEM((bpw, _D), jnp.float32),
            pltpu.SemaphoreType.DMA,
        ],
        compiler_params=pltpu.CompilerParams(use_tc_tiling_on_sc=False),
    )
    def od_gather(ori_hbm, dest_hbm, table_hbm, out_o_hbm, out_d_hbm,
                  idx_o, idx_d, rows_o, rows_d, sem):
        wid = lax.axis_index("s") * info.num_cores + lax.axis_index("c")
        base = wid * bpw
        pltpu.sync_copy(ori_hbm.at[pl.ds(base, bpw)], idx_o)
        pltpu.sync_copy(dest_hbm.at[pl.ds(base, bpw)], idx_d)
        copies = []
        for j in range(nch):
            sl = pl.ds(j * _CHUNK, _CHUNK)
            copies.append(
                pltpu.async_copy(table_hbm.at[idx_o.at[sl]], rows_o.at[sl], sem))
            copies.append(
                pltpu.async_copy(table_hbm.at[idx_d.at[sl]], rows_d.at[sl], sem))
        for c in copies:
            c.wait()
        pltpu.sync_copy(rows_o, out_o_hbm.at[pl.ds(base, bpw)])
        pltpu.sync_copy(rows_d, out_d_hbm.at[pl.ds(base, bpw)])

    return od_gather


def kernel(ori, dest, table):
    emb_o, emb_d = _build(ori.shape[0])(ori, dest, table)
    return (emb_o, emb_d)


# speedup vs baseline: 1.0004x; 1.0004x over previous
import functools

import jax
import jax.numpy as jnp
from jax import lax
from jax.experimental import pallas as pl
from jax.experimental.pallas import tpu as pltpu
from jax.experimental.pallas import tpu_sc as plsc

_D = 64
_CHUNK = 128


@functools.lru_cache(maxsize=None)
def _build(batch: int):
    info = plsc.get_sparse_core_info()
    nw = info.num_cores * info.num_subcores
    bpw = batch // nw
    nch = bpw // _CHUNK
    mesh = plsc.VectorSubcoreMesh(core_axis_name="c", subcore_axis_name="s")

    @functools.partial(
        pl.kernel,
        mesh=mesh,
        out_type=(
            jax.ShapeDtypeStruct((batch, _D), jnp.float32),
            jax.ShapeDtypeStruct((batch, _D), jnp.float32),
        ),
        scratch_types=[
            pltpu.VMEM((bpw,), jnp.int32),
            pltpu.VMEM((bpw,), jnp.int32),
            pltpu.VMEM((bpw, _D), jnp.float32),
            pltpu.VMEM((bpw, _D), jnp.float32),
            pltpu.SemaphoreType.DMA,
        ],
        compiler_params=pltpu.CompilerParams(use_tc_tiling_on_sc=False),
    )
    def od_gather(ori_hbm, dest_hbm, table_hbm, out_o_hbm, out_d_hbm,
                  idx_o, idx_d, rows_o, rows_d, sem):
        wid = lax.axis_index("s") * info.num_cores + lax.axis_index("c")
        base = wid * bpw
        pltpu.sync_copy(ori_hbm.at[pl.ds(base, bpw)], idx_o)
        pltpu.sync_copy(dest_hbm.at[pl.ds(base, bpw)], idx_d)
        copies = []
        for j in range(nch):
            sl = pl.ds(j * _CHUNK, _CHUNK)
            copies.append(
                pltpu.async_copy(table_hbm.at[idx_o.at[sl]], rows_o.at[sl], sem))
            copies.append(
                pltpu.async_copy(table_hbm.at[idx_d.at[sl]], rows_d.at[sl], sem))
        for c in copies:
            c.wait()
        pltpu.sync_copy(rows_o, out_o_hbm.at[pl.ds(base, bpw)])
        pltpu.sync_copy(rows_d, out_d_hbm.at[pl.ds(base, bpw)])

    return od_gather


def kernel(ori, dest, table):
    batch, = ori.shape
    return tuple(_build(batch)(ori, dest, table))


# speedup vs baseline: 1.6924x; 1.6918x over previous
import functools

import jax
import jax.numpy as jnp
from jax import lax
from jax.experimental import pallas as pl
from jax.experimental.pallas import tpu as pltpu
from jax.experimental.pallas import tpu_sc as plsc

_D = 64
_P = 256


@functools.lru_cache(maxsize=None)
def _build(batch: int):
    info = plsc.get_sparse_core_info()
    nw = info.num_cores * info.num_subcores
    bpw = batch // nw
    npass = bpw // _P
    mesh = plsc.VectorSubcoreMesh(core_axis_name="c", subcore_axis_name="s")

    @functools.partial(
        pl.kernel,
        mesh=mesh,
        out_type=(
            jax.ShapeDtypeStruct((batch, _D), jnp.float32),
            jax.ShapeDtypeStruct((batch, _D), jnp.float32),
        ),
        scratch_types=[
            pltpu.VMEM((bpw,), jnp.int32),
            pltpu.VMEM((bpw,), jnp.int32),
            pltpu.VMEM((_P, _D), jnp.float32),
            pltpu.VMEM((_P, _D), jnp.float32),
            pltpu.SemaphoreType.DMA,
        ],
    )
    def od_gather(ori_hbm, dest_hbm, table_hbm, out_o_hbm, out_d_hbm,
                  idx_o, idx_d, rows_o, rows_d, sem):
        wid = lax.axis_index("s") * info.num_cores + lax.axis_index("c")
        base = wid * bpw
        pltpu.sync_copy(ori_hbm.at[pl.ds(base, bpw)], idx_o)
        pltpu.sync_copy(dest_hbm.at[pl.ds(base, bpw)], idx_d)

        for p in range(npass):
            off = p * _P

            def issue(g, _, idx_ref=None, rows_ref=None):
                vec = idx_ref[pl.ds(off + g * 16, 16)]
                for l in range(16):
                    pltpu.async_copy(
                        table_hbm.at[vec[l]], rows_ref.at[g * 16 + l], sem)
                return _

            lax.fori_loop(
                0, _P // 16,
                functools.partial(issue, idx_ref=idx_o, rows_ref=rows_o), 0)
            lax.fori_loop(
                0, _P // 16,
                functools.partial(issue, idx_ref=idx_d, rows_ref=rows_d), 0)
            pltpu.make_async_copy(
                table_hbm.at[pl.ds(0, _P)], rows_o, sem).wait()
            pltpu.make_async_copy(
                table_hbm.at[pl.ds(0, _P)], rows_d, sem).wait()
            pltpu.sync_copy(rows_o, out_o_hbm.at[pl.ds(base + off, _P)])
            pltpu.sync_copy(rows_d, out_d_hbm.at[pl.ds(base + off, _P)])

    return od_gather


def kernel(ori, dest, table):
    batch, = ori.shape
    return tuple(_build(batch)(ori, dest, table))
